# baseline (speedup 1.0000x reference)
import jax
import jax.numpy as jnp
from jax.experimental import pallas as pl
from jax.experimental.pallas import tpu as pltpu

_MIN_DISTANCE = 3
_THRESHOLD_REL = 0.01
_MAX_NUM_PEAKS = 100
_H = 512
_W = 512
_BIG = 1 << 30


def _nms_channel_kernel(x_ref, scores_ref, gidx_ref, candT_ref):
    x = x_ref[0, 0]

    m = jnp.max(x)
    e = jnp.exp(x - m)
    s = jnp.sum(e)
    p = e / s

    k = 2 * _MIN_DISTANCE + 1
    zpad_r = jnp.zeros((_MIN_DISTANCE, _W), jnp.float32)
    pv = jnp.concatenate([zpad_r, p, zpad_r], axis=0)
    pooled_v = pv[0:_H, :]
    for d in range(1, k):
        pooled_v = jnp.maximum(pooled_v, pv[d:d + _H, :])
    zpad_c = jnp.zeros((_H, _MIN_DISTANCE), jnp.float32)
    ph = jnp.concatenate([zpad_c, pooled_v, zpad_c], axis=1)
    pooled = ph[:, 0:_W]
    for d in range(1, k):
        pooled = jnp.maximum(pooled, ph[:, d:d + _W])

    thr_abs = 1.0 / (_H * _W) * 2.0
    mx = jnp.max(p)
    mask = (pooled == p) & (p > thr_abs) & (p > _THRESHOLD_REL * mx)
    cand = jnp.where(mask, p, jnp.float32(-1.0))

    rows2d = jax.lax.broadcasted_iota(jnp.int32, (_H, _W), 0)
    cmax = jnp.max(cand, axis=0, keepdims=True)
    carg = jnp.min(jnp.where(cand == cmax, rows2d, _BIG), axis=0,
                   keepdims=True)

    candT_ref[...] = cand.T

    lane_w = jax.lax.broadcasted_iota(jnp.int32, (1, _W), 1)
    lane_k = jax.lax.broadcasted_iota(jnp.int32, (1, 128), 1)

    def body(i, st):
        cmax, carg, svec, gvec = st
        mval = jnp.max(cmax)
        g = jnp.min(jnp.where(cmax == mval, carg * _W + lane_w, _BIG))
        r = g // _W
        c = g % _W
        svec = jnp.where(lane_k == i, mval, svec)
        gvec = jnp.where(lane_k == i, g, gvec)
        rowv = candT_ref[pl.ds(c, 1), :]
        rowv = jnp.where(lane_w == r, jnp.float32(-3.0), rowv)
        candT_ref[pl.ds(c, 1), :] = rowv
        nm = jnp.max(rowv)
        na = jnp.min(jnp.where(rowv == nm, lane_w, _BIG))
        cmax = jnp.where(lane_w == c, nm, cmax)
        carg = jnp.where(lane_w == c, na, carg)
        return cmax, carg, svec, gvec

    svec0 = jnp.zeros((1, 128), jnp.float32)
    gvec0 = jnp.zeros((1, 128), jnp.int32)
    _, _, svec, gvec = jax.lax.fori_loop(
        0, _MAX_NUM_PEAKS, body, (cmax, carg, svec0, gvec0))

    scores_ref[0] = svec
    gidx_ref[0] = gvec


def kernel(belive_map):
    B, S, H, W = belive_map.shape
    bs = B * S
    raw_scores, raw_gidx = pl.pallas_call(
        _nms_channel_kernel,
        grid=(bs,),
        in_specs=[pl.BlockSpec((1, 1, H, W), lambda i: (i // S, i % S, 0, 0))],
        out_specs=[
            pl.BlockSpec((1, 1, 128), lambda i: (i, 0, 0)),
            pl.BlockSpec((1, 1, 128), lambda i: (i, 0, 0)),
        ],
        out_shape=[
            jax.ShapeDtypeStruct((bs, 1, 128), jnp.float32),
            jax.ShapeDtypeStruct((bs, 1, 128), jnp.int32),
        ],
        scratch_shapes=[pltpu.VMEM((W, H), jnp.float32)],
        compiler_params=pltpu.CompilerParams(
            dimension_semantics=("parallel",)),
    )(belive_map)

    scores_raw = raw_scores[:, 0, :_MAX_NUM_PEAKS].reshape(B, S, _MAX_NUM_PEAKS)
    g = raw_gidx[:, 0, :_MAX_NUM_PEAKS].reshape(B, S, _MAX_NUM_PEAKS)
    valid = scores_raw > 0.0
    scores = jnp.where(valid, scores_raw, 0.0)
    rows = g // W
    cols = g % W
    seg = jnp.broadcast_to(jnp.arange(S, dtype=jnp.int32)[None, :, None],
                           (B, S, _MAX_NUM_PEAKS))
    skeletons = jnp.stack([seg, cols, rows], axis=-1)
    return skeletons, scores, valid


# speedup vs baseline: 4.5326x; 4.5326x over previous
import jax
import jax.numpy as jnp
from jax.experimental import pallas as pl
from jax.experimental.pallas import tpu as pltpu

_MIN_DISTANCE = 3
_THRESHOLD_REL = 0.01
_MAX_NUM_PEAKS = 100
_H = 512
_W = 512
_C = 5
_BIG = 1 << 30


def _nms_kernel(x_ref, scores_ref, gidx_ref, candT_ref):
    x = x_ref[0]

    m = jnp.max(x, axis=(1, 2), keepdims=True)
    e = jnp.exp(x - m)
    s = jnp.sum(e, axis=(1, 2), keepdims=True)
    p = e / s

    k = 2 * _MIN_DISTANCE + 1
    zr = jnp.zeros((_C, _MIN_DISTANCE, _W), jnp.float32)
    pv = jnp.concatenate([zr, p, zr], axis=1)
    pooled_v = pv[:, 0:_H, :]
    for d in range(1, k):
        pooled_v = jnp.maximum(pooled_v, pv[:, d:d + _H, :])
    zc = jnp.zeros((_C, _H, _MIN_DISTANCE), jnp.float32)
    ph = jnp.concatenate([zc, pooled_v, zc], axis=2)
    pooled = ph[:, :, 0:_W]
    for d in range(1, k):
        pooled = jnp.maximum(pooled, ph[:, :, d:d + _W])

    thr_abs = 1.0 / (_H * _W) * 2.0
    mx = jnp.max(p, axis=(1, 2), keepdims=True)
    mask = (pooled == p) & (p > thr_abs) & (p > _THRESHOLD_REL * mx)
    cand = jnp.where(mask, p, jnp.float32(-1.0))

    rows3 = jax.lax.broadcasted_iota(jnp.int32, (_C, _H, _W), 1)
    cmax = jnp.max(cand, axis=1)
    carg = jnp.min(jnp.where(cand == cmax[:, None, :], rows3, _BIG),
                   axis=1)

    candT_ref[...] = jnp.swapaxes(cand, 1, 2)

    lane_w = jax.lax.broadcasted_iota(jnp.int32, (_C, _W), 1)
    lane_r = jax.lax.broadcasted_iota(jnp.int32, (_C, 1, _H), 2)
    lane_k = jax.lax.broadcasted_iota(jnp.int32, (_C, 128), 1)

    def body(i, st):
        cmax, carg, svec, gvec = st
        mval = jnp.max(cmax, axis=1, keepdims=True)
        g = jnp.min(jnp.where(cmax == mval, carg * _W + lane_w, _BIG),
                    axis=1, keepdims=True)
        svec = jnp.where(lane_k == i, mval, svec)
        gvec = jnp.where(lane_k == i, g, gvec)
        rowsbuf = jnp.concatenate(
            [candT_ref[pl.ds(ch, 1), pl.ds(g[ch, 0] % _W, 1), :]
             for ch in range(_C)], axis=0)
        rvec = (g // _W)[:, :, None]
        rowsbuf = jnp.where(lane_r == rvec, jnp.float32(-3.0), rowsbuf)
        for ch in range(_C):
            candT_ref[pl.ds(ch, 1), pl.ds(g[ch, 0] % _W, 1), :] = (
                rowsbuf[ch:ch + 1])
        nm = jnp.max(rowsbuf, axis=2)
        na = jnp.min(jnp.where(rowsbuf == nm[:, :, None], lane_r, _BIG),
                     axis=2)
        cvec = g % _W
        cmax = jnp.where(lane_w == cvec, nm, cmax)
        carg = jnp.where(lane_w == cvec, na, carg)
        return cmax, carg, svec, gvec

    svec0 = jnp.zeros((_C, 128), jnp.float32)
    gvec0 = jnp.zeros((_C, 128), jnp.int32)
    _, _, svec, gvec = jax.lax.fori_loop(
        0, _MAX_NUM_PEAKS, body, (cmax, carg, svec0, gvec0))

    scores_ref[0] = svec
    gidx_ref[0] = gvec


def kernel(belive_map):
    B, S, H, W = belive_map.shape
    nj = S // _C
    nprog = B * nj
    raw_scores, raw_gidx = pl.pallas_call(
        _nms_kernel,
        grid=(nprog,),
        in_specs=[pl.BlockSpec((1, _C, H, W),
                               lambda i: (i // nj, i % nj, 0, 0))],
        out_specs=[
            pl.BlockSpec((1, _C, 128), lambda i: (i, 0, 0)),
            pl.BlockSpec((1, _C, 128), lambda i: (i, 0, 0)),
        ],
        out_shape=[
            jax.ShapeDtypeStruct((nprog, _C, 128), jnp.float32),
            jax.ShapeDtypeStruct((nprog, _C, 128), jnp.int32),
        ],
        scratch_shapes=[pltpu.VMEM((_C, W, H), jnp.float32)],
        compiler_params=pltpu.CompilerParams(
            dimension_semantics=("arbitrary",)),
    )(belive_map)

    scores_raw = raw_scores.reshape(B * S, 128)[:, :_MAX_NUM_PEAKS]
    scores_raw = scores_raw.reshape(B, S, _MAX_NUM_PEAKS)
    g = raw_gidx.reshape(B * S, 128)[:, :_MAX_NUM_PEAKS]
    g = g.reshape(B, S, _MAX_NUM_PEAKS)
    valid = scores_raw > 0.0
    scores = jnp.where(valid, scores_raw, 0.0)
    rows = g // W
    cols = g % W
    seg = jnp.broadcast_to(jnp.arange(S, dtype=jnp.int32)[None, :, None],
                           (B, S, _MAX_NUM_PEAKS))
    skeletons = jnp.stack([seg, cols, rows], axis=-1)
    return skeletons, scores, valid


# speedup vs baseline: 9.2508x; 2.0410x over previous
import jax
import jax.numpy as jnp
from jax.experimental import pallas as pl
from jax.experimental.pallas import tpu as pltpu

_MIN_DISTANCE = 3
_THRESHOLD_REL = 0.01
_MAX_NUM_PEAKS = 100
_H = 512
_W = 512
_C = 10
_BIG = 1 << 30


def _pool7(padded):
    n = padded.shape[1] - 6
    t2 = jnp.maximum(padded[:, :n + 5], padded[:, 1:])
    t4 = jnp.maximum(t2[:, :n + 3], t2[:, 2:])
    return jnp.maximum(t4[:, :n], t4[:, 3:])


def _nms_kernel(x_ref, scores_ref, gidx_ref, candT_ref):
    x = x_ref[...]

    m = jnp.max(x, axis=(1, 2), keepdims=True)
    e = jnp.exp(x - m)
    s = jnp.sum(e, axis=(1, 2), keepdims=True)
    p = e / s

    zr = jnp.zeros((_C, _MIN_DISTANCE, _W), jnp.float32)
    pooled_v = _pool7(jnp.concatenate([zr, p, zr], axis=1))
    pT = jnp.swapaxes(p, 1, 2)
    pvT = jnp.swapaxes(pooled_v, 1, 2)
    zc = jnp.zeros((_C, _MIN_DISTANCE, _H), jnp.float32)
    pooledT = _pool7(jnp.concatenate([zc, pvT, zc], axis=1))

    thr_abs = 1.0 / (_H * _W) * 2.0
    mx = jnp.max(pT, axis=(1, 2), keepdims=True)
    maskT = (pooledT == pT) & (pT > thr_abs) & (pT > _THRESHOLD_REL * mx)
    candT = jnp.where(maskT, pT, jnp.float32(-1.0))
    candT_ref[...] = candT

    rowsT = jax.lax.broadcasted_iota(jnp.int32, (_C, _W, _H), 2)
    cmax = jnp.max(candT, axis=2)
    carg = jnp.min(jnp.where(candT == cmax[:, :, None], rowsT, _BIG),
                   axis=2)

    lane_w = jax.lax.broadcasted_iota(jnp.int32, (_C, _W), 1)
    lane_h = jax.lax.broadcasted_iota(jnp.int32, (_C, _H), 1)
    lane_k = jax.lax.broadcasted_iota(jnp.int32, (_C, 128), 1)

    def body(i, st):
        cmax, carg, svec, gvec = st
        mval = jnp.max(cmax, axis=1, keepdims=True)
        g = jnp.min(jnp.where(cmax == mval, carg * _W + lane_w, _BIG),
                    axis=1, keepdims=True)
        svec = jnp.where(lane_k == i, mval, svec)
        gvec = jnp.where(lane_k == i, g, gvec)
        rowsbuf = jnp.concatenate(
            [candT_ref[pl.ds(ch, 1), pl.ds(g[ch, 0] % _W, 1), :]
             .reshape(1, _H) for ch in range(_C)], axis=0)
        rowsbuf = jnp.where(lane_h == g // _W, jnp.float32(-3.0), rowsbuf)
        for ch in range(_C):
            candT_ref[pl.ds(ch, 1), pl.ds(g[ch, 0] % _W, 1), :] = (
                rowsbuf[ch:ch + 1].reshape(1, 1, _H))
        nm = jnp.max(rowsbuf, axis=1, keepdims=True)
        na = jnp.min(jnp.where(rowsbuf == nm, lane_h, _BIG),
                     axis=1, keepdims=True)
        cvec = g % _W
        cmax = jnp.where(lane_w == cvec, nm, cmax)
        carg = jnp.where(lane_w == cvec, na, carg)
        return cmax, carg, svec, gvec

    svec0 = jnp.zeros((_C, 128), jnp.float32)
    gvec0 = jnp.zeros((_C, 128), jnp.int32)
    _, _, svec, gvec = jax.lax.fori_loop(
        0, _MAX_NUM_PEAKS, body, (cmax, carg, svec0, gvec0))

    scores_ref[0] = svec
    gidx_ref[0] = gvec


def kernel(belive_map):
    B, S, H, W = belive_map.shape
    bs = B * S
    nprog = bs // _C
    xflat = belive_map.reshape(bs, H, W)
    raw_scores, raw_gidx = pl.pallas_call(
        _nms_kernel,
        grid=(nprog,),
        in_specs=[pl.BlockSpec((_C, H, W), lambda i: (i, 0, 0))],
        out_specs=[
            pl.BlockSpec((1, _C, 128), lambda i: (i, 0, 0)),
            pl.BlockSpec((1, _C, 128), lambda i: (i, 0, 0)),
        ],
        out_shape=[
            jax.ShapeDtypeStruct((nprog, _C, 128), jnp.float32),
            jax.ShapeDtypeStruct((nprog, _C, 128), jnp.int32),
        ],
        scratch_shapes=[pltpu.VMEM((_C, W, H), jnp.float32)],
        compiler_params=pltpu.CompilerParams(
            dimension_semantics=("arbitrary",)),
    )(xflat)

    scores_raw = raw_scores.reshape(bs, 128)[:, :_MAX_NUM_PEAKS]
    scores_raw = scores_raw.reshape(B, S, _MAX_NUM_PEAKS)
    g = raw_gidx.reshape(bs, 128)[:, :_MAX_NUM_PEAKS].reshape(B, S,
                                                              _MAX_NUM_PEAKS)
    valid = scores_raw > 0.0
    scores = jnp.where(valid, scores_raw, 0.0)
    rows = g // W
    cols = g % W
    seg = jnp.broadcast_to(jnp.arange(S, dtype=jnp.int32)[None, :, None],
                           (B, S, _MAX_NUM_PEAKS))
    skeletons = jnp.stack([seg, cols, rows], axis=-1)
    return skeletons, scores, valid
